# Initial kernel scaffold; baseline (speedup 1.0000x reference)
import functools

import jax
import jax.numpy as jnp
from jax import lax
from jax.experimental import pallas as pl
from jax.experimental.pallas import tpu as pltpu
from jax.experimental.pallas import tpu_sc as plsc

B = 4096
L = 20
K = 64
NC = 2
NS = 16
NW = NC * NS
BPW = B // NW
G = (BPW * L) // 128

_mesh = plsc.VectorSubcoreMesh(core_axis_name="c", subcore_axis_name="s")


@functools.partial(
    pl.kernel,
    mesh=_mesh,
    out_type=jax.ShapeDtypeStruct((24, B, K), jnp.float32),
    scratch_types=[
        pltpu.VMEM((G, 128), jnp.int32),
        pltpu.VMEM((G, 128), jnp.int32),
        pltpu.VMEM((9, 128), jnp.int32),
        pltpu.VMEM((128, K), jnp.float32),
        pltpu.VMEM_SHARED((NS * BPW, K), jnp.float32),
    ],
)
def _sc_gather(IL1, IL2, IL3, IL4, LI1, LI2, LI3, LI4, UI1, UI2, UI3, UI4, IU,
               uidh, iid1, iid2, iid3, iid4, neg1, neg2, neg3, neg4,
               bk1, bk2, bk3, bk4, path, zrh,
               out, bkv, patv, idxv, gbuf, shacc):
    wid = lax.axis_index("s") * NC + lax.axis_index("c")
    sid = lax.axis_index("s")
    base = wid * BPW

    ILs = (IL1, IL2, IL3, IL4)
    LIs = (LI1, LI2, LI3, LI4)
    UIs = (UI1, UI2, UI3, UI4)
    iids = (iid1, iid2, iid3, iid4)
    negs = (neg1, neg2, neg3, neg4)
    bks = (bk1, bk2, bk3, bk4)

    pltpu.sync_copy(path.at[sid], patv)
    pltpu.sync_copy(uidh.at[wid], idxv.at[0])
    for i in range(4):
        pltpu.sync_copy(iids[i].at[wid], idxv.at[1 + i])
        pltpu.sync_copy(negs[i].at[wid], idxv.at[5 + i])

    plan = []
    for i in range(4):
        plan.append((ILs[i], 1 + i, 4 + i))
        plan.append((ILs[i], 5 + i, 8 + i))
        plan.append((UIs[i], 0, 12 + i))
        plan.append((IU, 1 + i, 16 + i))
        plan.append((IU, 5 + i, 20 + i))
    for tbl, r, s in plan:
        pltpu.sync_copy(tbl.at[idxv.at[r]], gbuf)
        pltpu.sync_copy(gbuf, out.at[s, pl.ds(base, BPW)])

    for i in range(4):
        pltpu.sync_copy(bks[i].at[wid], bkv)
        pltpu.sync_copy(zrh, shacc.at[pl.ds(sid * BPW, BPW)])

        @pl.loop(0, G)
        def _(g):
            pltpu.sync_copy(LIs[i].at[bkv.at[g]], gbuf)
            pltpu.sync_copy(gbuf, shacc.at[patv.at[g]], add=True)

        pltpu.sync_copy(shacc.at[pl.ds(sid * BPW, BPW)],
                        out.at[i, pl.ds(base, BPW)])


_BB = 512


def _tc_body(x_ref, al_ref, o_ref):
    x = x_ref[...]
    a4 = jax.nn.sigmoid(al_ref[0, :])
    fmc = [x[t] * jnp.float32(1.0 / L) for t in range(4)]
    neg_inf = jnp.float32(-2.0 ** 32 + 1)
    for i in range(4):
        u = x[12 + i]
        for sgn in range(2):
            e = x[4 + 4 * sgn + i]
            iu = x[16 + 4 * sgn + i]
            d = jnp.concatenate(
                [jnp.sum(fmc[t] * e, axis=1, keepdims=True) for t in range(4)],
                axis=1)
            w = d * jnp.float32(0.125)
            w = jnp.where(w == 0.0, neg_inf, w)
            w = w - jnp.max(w, axis=1, keepdims=True)
            p = jnp.exp(w)
            p = p / jnp.sum(p, axis=1, keepdims=True)
            att = jnp.sum(p * d, axis=1)
            mf = jnp.sum(u * iu, axis=1)
            o_ref[2 * i + sgn, :] = a4[i] * att + (1.0 - a4[i]) * mf


def kernel(uid, basket_1, basket_2, basket_3, basket_4,
           iid_1, iid_2, iid_3, iid_4,
           neg_iid_1, neg_iid_2, neg_iid_3, neg_iid_4,
           IL_1, IL_2, IL_3, IL_4, LI_1, LI_2, LI_3, LI_4,
           UI_1, UI_2, UI_3, UI_4, IU,
           alpha_mor, alpha_aft, alpha_eve, alpha_deep):
    i32 = jnp.int32
    uidh = uid.astype(i32).reshape(NW, BPW)
    iids = [x.astype(i32).reshape(NW, BPW)
            for x in (iid_1, iid_2, iid_3, iid_4)]
    negs = [x.astype(i32).reshape(NW, BPW)
            for x in (neg_iid_1, neg_iid_2, neg_iid_3, neg_iid_4)]
    bks = [x.astype(i32).reshape(NW, G, 128)
           for x in (basket_1, basket_2, basket_3, basket_4)]
    pat = (jnp.arange(BPW * L, dtype=i32) // L).reshape(G, 128)
    path = pat[None, :, :] + (jnp.arange(NS, dtype=i32) * BPW)[:, None, None]
    zrh = jnp.zeros((BPW, K), jnp.float32)

    gath = _sc_gather(IL_1, IL_2, IL_3, IL_4, LI_1, LI_2, LI_3, LI_4,
                      UI_1, UI_2, UI_3, UI_4, IU,
                      uidh, *iids, *negs, *bks, path, zrh)

    alphas = jnp.stack([alpha_mor, alpha_aft, alpha_eve, alpha_deep])
    alphas = alphas.astype(jnp.float32).reshape(1, 4)

    out = pl.pallas_call(
        _tc_body,
        grid=(B // _BB,),
        in_specs=[
            pl.BlockSpec((24, _BB, K), lambda j: (0, j, 0)),
            pl.BlockSpec((1, 4), lambda j: (0, 0)),
        ],
        out_specs=pl.BlockSpec((8, _BB), lambda j: (0, j)),
        out_shape=jax.ShapeDtypeStruct((8, B), jnp.float32),
    )(gath, alphas)

    return tuple(out[i] for i in range(8))


# speedup vs baseline: 1.0281x; 1.0281x over previous
import functools

import jax
import jax.numpy as jnp
from jax import lax
from jax.experimental import pallas as pl
from jax.experimental.pallas import tpu as pltpu
from jax.experimental.pallas import tpu_sc as plsc

B = 4096
L = 20
K = 64
NC = 2
NS = 16
NW = NC * NS
BPW = B // NW
G = (BPW * L) // 128

_mesh = plsc.VectorSubcoreMesh(core_axis_name="c", subcore_axis_name="s")


@functools.partial(
    pl.kernel,
    mesh=_mesh,
    compiler_params=pltpu.CompilerParams(use_tc_tiling_on_sc=False),
    out_type=jax.ShapeDtypeStruct((24, B, K), jnp.float32),
    scratch_types=[
        pltpu.VMEM((G, 128), jnp.int32),
        pltpu.VMEM((G, 128), jnp.int32),
        pltpu.VMEM((9, 128), jnp.int32),
        pltpu.VMEM((128, K), jnp.float32),
        pltpu.VMEM_SHARED((NS * BPW, K), jnp.float32),
    ],
)
def _sc_gather(IL1, IL2, IL3, IL4, LI1, LI2, LI3, LI4, UI1, UI2, UI3, UI4, IU,
               uidh, iid1, iid2, iid3, iid4, neg1, neg2, neg3, neg4,
               bk1, bk2, bk3, bk4, path, zrh,
               out, bkv, patv, idxv, gbuf, shacc):
    wid = lax.axis_index("s") * NC + lax.axis_index("c")
    sid = lax.axis_index("s")
    base = wid * BPW

    ILs = (IL1, IL2, IL3, IL4)
    LIs = (LI1, LI2, LI3, LI4)
    UIs = (UI1, UI2, UI3, UI4)
    iids = (iid1, iid2, iid3, iid4)
    negs = (neg1, neg2, neg3, neg4)
    bks = (bk1, bk2, bk3, bk4)

    pltpu.sync_copy(path.at[sid], patv)
    pltpu.sync_copy(uidh.at[wid], idxv.at[0])
    for i in range(4):
        pltpu.sync_copy(iids[i].at[wid], idxv.at[1 + i])
        pltpu.sync_copy(negs[i].at[wid], idxv.at[5 + i])

    plan = []
    for i in range(4):
        plan.append((ILs[i], 1 + i, 4 + i))
        plan.append((ILs[i], 5 + i, 8 + i))
        plan.append((UIs[i], 0, 12 + i))
        plan.append((IU, 1 + i, 16 + i))
        plan.append((IU, 5 + i, 20 + i))
    for tbl, r, s in plan:
        pltpu.sync_copy(tbl.at[idxv.at[r]], gbuf)
        pltpu.sync_copy(gbuf, out.at[s, pl.ds(base, BPW)])

    for i in range(4):
        pltpu.sync_copy(bks[i].at[wid], bkv)
        pltpu.sync_copy(zrh, shacc.at[pl.ds(sid * BPW, BPW)])

        @pl.loop(0, G)
        def _(g):
            pltpu.sync_copy(LIs[i].at[bkv.at[g]], gbuf)
            pltpu.sync_copy(gbuf, shacc.at[patv.at[g]], add=True)

        pltpu.sync_copy(shacc.at[pl.ds(sid * BPW, BPW)],
                        out.at[i, pl.ds(base, BPW)])


_BB = 512


def _tc_body(x_ref, al_ref, o_ref):
    x = x_ref[...]
    a4 = jax.nn.sigmoid(al_ref[0, :])
    fmc = [x[t] * jnp.float32(1.0 / L) for t in range(4)]
    neg_inf = jnp.float32(-2.0 ** 32 + 1)
    for i in range(4):
        u = x[12 + i]
        for sgn in range(2):
            e = x[4 + 4 * sgn + i]
            iu = x[16 + 4 * sgn + i]
            d = jnp.concatenate(
                [jnp.sum(fmc[t] * e, axis=1, keepdims=True) for t in range(4)],
                axis=1)
            w = d * jnp.float32(0.125)
            w = jnp.where(w == 0.0, neg_inf, w)
            w = w - jnp.max(w, axis=1, keepdims=True)
            p = jnp.exp(w)
            p = p / jnp.sum(p, axis=1, keepdims=True)
            att = jnp.sum(p * d, axis=1)
            mf = jnp.sum(u * iu, axis=1)
            o_ref[2 * i + sgn, :] = a4[i] * att + (1.0 - a4[i]) * mf


def kernel(uid, basket_1, basket_2, basket_3, basket_4,
           iid_1, iid_2, iid_3, iid_4,
           neg_iid_1, neg_iid_2, neg_iid_3, neg_iid_4,
           IL_1, IL_2, IL_3, IL_4, LI_1, LI_2, LI_3, LI_4,
           UI_1, UI_2, UI_3, UI_4, IU,
           alpha_mor, alpha_aft, alpha_eve, alpha_deep):
    i32 = jnp.int32
    uidh = uid.astype(i32).reshape(NW, BPW)
    iids = [x.astype(i32).reshape(NW, BPW)
            for x in (iid_1, iid_2, iid_3, iid_4)]
    negs = [x.astype(i32).reshape(NW, BPW)
            for x in (neg_iid_1, neg_iid_2, neg_iid_3, neg_iid_4)]
    bks = [x.astype(i32).reshape(NW, G, 128)
           for x in (basket_1, basket_2, basket_3, basket_4)]
    pat = (jnp.arange(BPW * L, dtype=i32) // L).reshape(G, 128)
    path = pat[None, :, :] + (jnp.arange(NS, dtype=i32) * BPW)[:, None, None]
    zrh = jnp.zeros((BPW, K), jnp.float32)

    gath = _sc_gather(IL_1, IL_2, IL_3, IL_4, LI_1, LI_2, LI_3, LI_4,
                      UI_1, UI_2, UI_3, UI_4, IU,
                      uidh, *iids, *negs, *bks, path, zrh)

    alphas = jnp.stack([alpha_mor, alpha_aft, alpha_eve, alpha_deep])
    alphas = alphas.astype(jnp.float32).reshape(1, 4)

    out = pl.pallas_call(
        _tc_body,
        grid=(B // _BB,),
        in_specs=[
            pl.BlockSpec((24, _BB, K), lambda j: (0, j, 0)),
            pl.BlockSpec((1, 4), lambda j: (0, 0)),
        ],
        out_specs=pl.BlockSpec((8, _BB), lambda j: (0, j)),
        out_shape=jax.ShapeDtypeStruct((8, B), jnp.float32),
    )(gath, alphas)

    return tuple(out[i] for i in range(8))


# speedup vs baseline: 1.1751x; 1.1430x over previous
import functools

import jax
import jax.numpy as jnp
from jax import lax
from jax.experimental import pallas as pl
from jax.experimental.pallas import tpu as pltpu
from jax.experimental.pallas import tpu_sc as plsc

B = 4096
L = 20
K = 64
NC = 2
NS = 16
NW = NC * NS
BPW = B // NW
G = (BPW * L) // 128

_mesh = plsc.VectorSubcoreMesh(core_axis_name="c", subcore_axis_name="s")
_sc_params = pltpu.CompilerParams(use_tc_tiling_on_sc=False)


@functools.partial(
    pl.kernel,
    mesh=_mesh,
    compiler_params=_sc_params,
    out_type=jax.ShapeDtypeStruct((B, K), jnp.float32),
    scratch_types=[
        pltpu.VMEM((G, 128), jnp.int32),
        pltpu.VMEM((G, 128), jnp.int32),
        pltpu.VMEM((128, K), jnp.float32),
        pltpu.VMEM((128, K), jnp.float32),
        pltpu.VMEM_SHARED((NS * BPW, K), jnp.float32),
        pltpu.SemaphoreType.DMA,
        pltpu.SemaphoreType.DMA,
    ],
)
def _sc_pool(LI, bk, path, zrh, out, bkv, patv, gbufa, gbufb, shacc, sga, sgb):
    wid = lax.axis_index("s") * NC + lax.axis_index("c")
    sid = lax.axis_index("s")
    base = wid * BPW
    bufs = (gbufa, gbufb)
    sems = (sga, sgb)

    pltpu.sync_copy(path.at[sid], patv)
    pltpu.sync_copy(bk.at[wid], bkv)
    pltpu.sync_copy(zrh, shacc.at[pl.ds(sid * BPW, BPW)])

    pend = [None, None]
    pend[0] = pltpu.async_copy(LI.at[bkv.at[0]], bufs[0], sems[0])
    for g in range(G):
        if g + 1 < G:
            pend[(g + 1) % 2] = pltpu.async_copy(
                LI.at[bkv.at[g + 1]], bufs[(g + 1) % 2], sems[(g + 1) % 2])
        pend[g % 2].wait()
        pltpu.sync_copy(bufs[g % 2], shacc.at[patv.at[g]], add=True)

    pltpu.sync_copy(shacc.at[pl.ds(sid * BPW, BPW)], out.at[pl.ds(base, BPW)])


@functools.partial(
    pl.kernel,
    mesh=_mesh,
    compiler_params=_sc_params,
    out_type=jax.ShapeDtypeStruct((20, B, K), jnp.float32),
    scratch_types=[
        pltpu.VMEM((9, 128), jnp.int32),
        pltpu.VMEM((128, K), jnp.float32),
        pltpu.VMEM((128, K), jnp.float32),
        pltpu.SemaphoreType.DMA,
        pltpu.SemaphoreType.DMA,
    ],
)
def _sc_rows(IL1, IL2, IL3, IL4, UI1, UI2, UI3, UI4, IU, idxh,
             out, idxv, gbufa, gbufb, sga, sgb):
    wid = lax.axis_index("s") * NC + lax.axis_index("c")
    base = wid * BPW
    bufs = (gbufa, gbufb)
    gsems = (sga, sgb)

    pltpu.sync_copy(idxh.at[wid], idxv)

    ILs = (IL1, IL2, IL3, IL4)
    UIs = (UI1, UI2, UI3, UI4)
    plan = []
    for i in range(4):
        plan.append((ILs[i], 1 + i, i))
        plan.append((ILs[i], 5 + i, 4 + i))
        plan.append((UIs[i], 0, 8 + i))
        plan.append((IU, 1 + i, 12 + i))
        plan.append((IU, 5 + i, 16 + i))

    n = len(plan)
    gp = [None, None]
    gp[0] = pltpu.async_copy(plan[0][0].at[idxv.at[plan[0][1]]], bufs[0],
                             gsems[0])
    for j in range(n):
        if j + 1 < n:
            tbl, r, _ = plan[j + 1]
            gp[(j + 1) % 2] = pltpu.async_copy(
                tbl.at[idxv.at[r]], bufs[(j + 1) % 2], gsems[(j + 1) % 2])
        gp[j % 2].wait()
        slot = plan[j][2]
        pltpu.sync_copy(bufs[j % 2], out.at[slot, pl.ds(base, BPW)])


_BB = 512


def _tc_body(f1_ref, f2_ref, f3_ref, f4_ref, r_ref, al_ref, o_ref):
    fmc = [f_ref[...] * jnp.float32(1.0 / L)
           for f_ref in (f1_ref, f2_ref, f3_ref, f4_ref)]
    r = r_ref[...]
    a4 = jax.nn.sigmoid(al_ref[0, :])
    neg_inf = jnp.float32(-2.0 ** 32 + 1)
    for i in range(4):
        u = r[8 + i]
        for sgn in range(2):
            e = r[4 * sgn + i]
            iu = r[12 + 4 * sgn + i]
            d = jnp.concatenate(
                [jnp.sum(fmc[t] * e, axis=1, keepdims=True) for t in range(4)],
                axis=1)
            w = d * jnp.float32(0.125)
            w = jnp.where(w == 0.0, neg_inf, w)
            w = w - jnp.max(w, axis=1, keepdims=True)
            p = jnp.exp(w)
            p = p / jnp.sum(p, axis=1, keepdims=True)
            att = jnp.sum(p * d, axis=1)
            mf = jnp.sum(u * iu, axis=1)
            o_ref[2 * i + sgn, :] = a4[i] * att + (1.0 - a4[i]) * mf


def kernel(uid, basket_1, basket_2, basket_3, basket_4,
           iid_1, iid_2, iid_3, iid_4,
           neg_iid_1, neg_iid_2, neg_iid_3, neg_iid_4,
           IL_1, IL_2, IL_3, IL_4, LI_1, LI_2, LI_3, LI_4,
           UI_1, UI_2, UI_3, UI_4, IU,
           alpha_mor, alpha_aft, alpha_eve, alpha_deep):
    i32 = jnp.int32
    idx_all = jnp.stack([uid, iid_1, iid_2, iid_3, iid_4,
                         neg_iid_1, neg_iid_2, neg_iid_3, neg_iid_4])
    idx_all = idx_all.astype(i32).reshape(9, NW, 128).transpose(1, 0, 2)
    bks = [x.astype(i32).reshape(NW, G, 128)
           for x in (basket_1, basket_2, basket_3, basket_4)]
    pat = (jnp.arange(BPW * L, dtype=i32) // L).reshape(G, 128)
    path = pat[None, :, :] + (jnp.arange(NS, dtype=i32) * BPW)[:, None, None]
    zrh = jnp.zeros((BPW, K), jnp.float32)

    fmcs = [_sc_pool(LIt, bkt, path, zrh)
            for LIt, bkt in zip((LI_1, LI_2, LI_3, LI_4), bks)]
    rows = _sc_rows(IL_1, IL_2, IL_3, IL_4, UI_1, UI_2, UI_3, UI_4, IU,
                    idx_all)

    alphas = jnp.stack([alpha_mor, alpha_aft, alpha_eve, alpha_deep])
    alphas = alphas.astype(jnp.float32).reshape(1, 4)

    fspec = pl.BlockSpec((_BB, K), lambda j: (j, 0))
    out = pl.pallas_call(
        _tc_body,
        grid=(B // _BB,),
        in_specs=[fspec, fspec, fspec, fspec,
                  pl.BlockSpec((20, _BB, K), lambda j: (0, j, 0)),
                  pl.BlockSpec((1, 4), lambda j: (0, 0))],
        out_specs=pl.BlockSpec((8, _BB), lambda j: (0, j)),
        out_shape=jax.ShapeDtypeStruct((8, B), jnp.float32),
    )(*fmcs, rows, alphas)

    return tuple(out[i] for i in range(8))
